# baseline (speedup 1.0000x reference)
import functools

import numpy as np
import jax
import jax.numpy as jnp
from jax.experimental import pallas as pl

_VOCAB = 1000
_SCHEDULE = {}


def _uniform_body(col, out_ref):
    _, bt, v = out_ref.shape
    lane = jax.lax.broadcasted_iota(jnp.int32, (1, 8, v), 2)
    rows8 = jnp.where(lane == col, 10.0, -10.0)
    out_ref[...] = jnp.broadcast_to(rows8[:, :1], (1, bt, v))


def _general_body(col_ref, out_ref):
    _, bt, v = out_ref.shape
    lane = jax.lax.broadcasted_iota(jnp.int32, (1, bt, v), 2)
    out_ref[...] = jnp.where(lane == col_ref[...][None], 10.0, -10.0)


def kernel(input_ids, anchor):
    B, T = input_ids.shape
    past_len = 0
    cols_np = np.array(
        [int(_SCHEDULE.get(past_len + t, 1)) for t in range(T)], dtype=np.int32
    )

    bt = 1024
    out_shape = jax.ShapeDtypeStruct((B, T, _VOCAB), jnp.float32)
    if bool((cols_np == cols_np[0]).all()):
        return pl.pallas_call(
            functools.partial(_uniform_body, int(cols_np[0])),
            grid=(B, T // bt),
            out_specs=pl.BlockSpec((1, bt, _VOCAB), lambda b, j: (b, j, 0)),
            out_shape=out_shape,
        )()
    cols = jnp.asarray(cols_np.reshape(T, 1))
    return pl.pallas_call(
        _general_body,
        grid=(B, T // bt),
        in_specs=[pl.BlockSpec((bt, 1), lambda b, j: (j, 0))],
        out_specs=pl.BlockSpec((1, bt, _VOCAB), lambda b, j: (b, j, 0)),
        out_shape=out_shape,
    )(cols)
